# baseline (speedup 1.0000x reference)
import functools

import jax
import jax.numpy as jnp
from jax import lax
from jax.experimental import pallas as pl
from jax.experimental.pallas import tpu as pltpu
from jax.experimental.pallas import tpu_sc as plsc

NUM_CLASSES = 1000000
EMBED_DIM = 64
BATCH = 16384

NC = 2
NS = 16
NW = NC * NS

B_PER_W = BATCH // NW
CHUNK = 128
NCHUNK = B_PER_W // CHUNK


@functools.partial(
    pl.kernel,
    mesh=plsc.VectorSubcoreMesh(core_axis_name="c", subcore_axis_name="s"),
    out_type=jax.ShapeDtypeStruct((BATCH, EMBED_DIM), jnp.float32),
    scratch_types=[
        pltpu.VMEM((NCHUNK, CHUNK), jnp.int32),
        pltpu.VMEM((B_PER_W, EMBED_DIM), jnp.float32),
        pltpu.SemaphoreType.DMA,
    ],
    compiler_params=pltpu.CompilerParams(use_tc_tiling_on_sc=False),
)
def _gather(table_hbm, idx_hbm, out_hbm, idx_v, rows_v, sem):
    wid = lax.axis_index("s") * NC + lax.axis_index("c")
    base = wid * B_PER_W
    pltpu.sync_copy(idx_hbm.at[wid], idx_v)
    copies = [
        pltpu.async_copy(
            table_hbm.at[idx_v.at[j]],
            rows_v.at[pl.ds(j * CHUNK, CHUNK)],
            sem,
        )
        for j in range(NCHUNK)
    ]
    for c in copies:
        c.wait()
    pltpu.sync_copy(rows_v, out_hbm.at[pl.ds(base, B_PER_W)])


def kernel(labels, embedding_table):
    idx = labels.astype(jnp.int32).reshape(NW, NCHUNK, CHUNK)
    return _gather(embedding_table, idx)


# speedup vs baseline: 1.7068x; 1.7068x over previous
import functools

import jax
import jax.numpy as jnp
from jax import lax
from jax.experimental import pallas as pl
from jax.experimental.pallas import tpu as pltpu
from jax.experimental.pallas import tpu_sc as plsc

NUM_CLASSES = 1000000
EMBED_DIM = 64
BATCH = 16384

NC = 2
NS = 16
NW = NC * NS

B_PER_W = BATCH // NW


@functools.partial(
    pl.kernel,
    mesh=plsc.VectorSubcoreMesh(core_axis_name="c", subcore_axis_name="s"),
    out_type=jax.ShapeDtypeStruct((BATCH, EMBED_DIM), jnp.float32),
    scratch_types=[
        pltpu.VMEM((B_PER_W,), jnp.int32),
        pltpu.VMEM((B_PER_W, EMBED_DIM), jnp.float32),
        pltpu.SemaphoreType.DMA,
    ],
)
def _gather(table_hbm, idx_hbm, out_hbm, idx_v, rows_v, sem):
    wid = lax.axis_index("s") * NC + lax.axis_index("c")
    base = wid * B_PER_W
    pltpu.sync_copy(idx_hbm.at[pl.ds(base, B_PER_W)], idx_v)

    def body(g, carry):
        vec = idx_v[pl.ds(g * 16, 16)]
        for lane in range(16):
            r = vec[lane]
            pltpu.async_copy(
                table_hbm.at[pl.ds(r, 1)],
                rows_v.at[pl.ds(g * 16 + lane, 1)],
                sem,
            )
        return carry

    lax.fori_loop(0, B_PER_W // 16, body, None)
    pltpu.make_async_copy(table_hbm.at[pl.ds(0, B_PER_W)], rows_v, sem).wait()
    pltpu.sync_copy(rows_v, out_hbm.at[pl.ds(base, B_PER_W)])


def kernel(labels, embedding_table):
    idx = labels.astype(jnp.int32)
    return _gather(embedding_table, idx)


# speedup vs baseline: 2.3866x; 1.3983x over previous
import functools

import jax
import jax.numpy as jnp
from jax import lax
from jax.experimental import pallas as pl
from jax.experimental.pallas import tpu as pltpu
from jax.experimental.pallas import tpu_sc as plsc

NUM_CLASSES = 1000000
EMBED_DIM = 64
BATCH = 16384

NC = 2
NS = 16
NW = NC * NS

B_PER_W = BATCH // NW
GRP = 16
NGRP = B_PER_W // GRP
LANES = 128


@functools.partial(
    pl.kernel,
    mesh=plsc.VectorSubcoreMesh(core_axis_name="c", subcore_axis_name="s"),
    out_type=jax.ShapeDtypeStruct((BATCH, EMBED_DIM), jnp.float32),
    scratch_types=[
        pltpu.VMEM((B_PER_W,), jnp.int32),
        pltpu.VMEM((8, EMBED_DIM, LANES), jnp.float32),
        pltpu.VMEM((2, GRP, EMBED_DIM), jnp.float32),
        pltpu.SemaphoreType.DMA,
        pltpu.SemaphoreType.DMA,
    ],
    compiler_params=pltpu.CompilerParams(needs_layout_passes=False),
)
def _gather_blocks(table_hbm, idx_hbm, out_hbm, idx_v, blk_v, grp_v, sem, out_sem):
    wid = lax.axis_index("s") * NC + lax.axis_index("c")
    base = wid * B_PER_W
    pltpu.sync_copy(idx_hbm.at[pl.ds(base, B_PER_W)], idx_v)

    row_ids = [jnp.arange(16, dtype=jnp.int32) + 16 * t for t in range(EMBED_DIM // 16)]

    def extract(vec, p, lo):
        for lane in range(8):
            j = lo + lane
            q = jnp.full((16,), vec[j] & 127, dtype=jnp.int32)
            blk = blk_v.at[lane]
            for t in range(EMBED_DIM // 16):
                vals = plsc.load_gather(blk, [row_ids[t], q])
                grp_v.at[p].at[j].at[pl.ds(t * 16, 16)][...] = vals

    def body(g, carry):
        vec = idx_v[pl.ds(g * GRP, GRP)]

        @pl.when(g >= 2)
        def _():
            pltpu.make_async_copy(
                out_hbm.at[pl.ds(base, GRP)], grp_v.at[0], out_sem
            ).wait()

        def wave(p, lo):
            copies = []
            for lane in range(8):
                k128 = pl.multiple_of((vec[lo + lane] >> 7) * 128, 128)
                copies.append(
                    pltpu.async_copy(
                        table_hbm.at[:, pl.ds(k128, LANES)],
                        blk_v.at[lane],
                        sem,
                    )
                )
            for c in copies:
                c.wait()
            extract(vec, p, lo)

        @pl.when(lax.rem(g, 2) == 0)
        def _():
            wave(0, 0)
            wave(0, 8)
            pltpu.async_copy(
                grp_v.at[0], out_hbm.at[pl.ds(base + g * GRP, GRP)], out_sem
            )

        @pl.when(lax.rem(g, 2) == 1)
        def _():
            wave(1, 0)
            wave(1, 8)
            pltpu.async_copy(
                grp_v.at[1], out_hbm.at[pl.ds(base + g * GRP, GRP)], out_sem
            )

        return carry

    lax.fori_loop(0, NGRP, body, None)
    for _ in range(2):
        pltpu.make_async_copy(
            out_hbm.at[pl.ds(base, GRP)], grp_v.at[0], out_sem
        ).wait()


def kernel(labels, embedding_table):
    return _gather_blocks(embedding_table.T, labels.astype(jnp.int32))


# speedup vs baseline: 3.3319x; 1.3961x over previous
import functools

import jax
import jax.numpy as jnp
from jax import lax
from jax.experimental import pallas as pl
from jax.experimental.pallas import tpu as pltpu
from jax.experimental.pallas import tpu_sc as plsc

NUM_CLASSES = 1000000
EMBED_DIM = 64
BATCH = 16384

NC = 2
NS = 16
NW = NC * NS

B_PER_W = BATCH // NW
GRP = 16
NGRP = B_PER_W // GRP
LANES = 128


@functools.partial(
    pl.kernel,
    mesh=plsc.VectorSubcoreMesh(core_axis_name="c", subcore_axis_name="s"),
    out_type=jax.ShapeDtypeStruct((BATCH, EMBED_DIM), jnp.float32),
    scratch_types=[
        pltpu.VMEM((B_PER_W,), jnp.int32),
        pltpu.VMEM((B_PER_W,), jnp.int32),
        pltpu.VMEM((8, EMBED_DIM, LANES), jnp.float32),
        pltpu.VMEM((2, GRP, EMBED_DIM), jnp.float32),
        pltpu.SemaphoreType.DMA,
        pltpu.SemaphoreType.DMA,
    ],
    compiler_params=pltpu.CompilerParams(needs_layout_passes=False),
)
def _gather_blocks(table_hbm, idx_hbm, ord_hbm, out_hbm,
                   idx_v, ord_v, blk_v, grp_v, sem, out_sem):
    wid = lax.axis_index("s") * NC + lax.axis_index("c")
    base = wid * B_PER_W
    pltpu.sync_copy(idx_hbm.at[pl.ds(base, B_PER_W)], idx_v)
    pltpu.sync_copy(ord_hbm.at[pl.ds(base, B_PER_W)], ord_v)

    lane_ids = jnp.arange(16, dtype=jnp.int32)
    row_ids = [lane_ids + 16 * t for t in range(EMBED_DIM // 16)]

    def body(g, carry):
        vec = idx_v[pl.ds(g * GRP, GRP)]
        ovec = ord_v[pl.ds(g * GRP, GRP)]

        kvec = vec >> 7
        pidx = jnp.maximum(g * GRP + lane_ids - 1, 0)
        prev = plsc.load_gather(idx_v, [pidx]) >> 7
        m = (kvec != prev) | (lane_ids == 0) | (lane_ids == 8)
        mi = m.astype(jnp.int32)
        starts = jnp.where(m, lane_ids, 0)
        slot_vec = plsc.cummax(starts) & 7

        @pl.when(g >= 2)
        def _():
            pltpu.make_async_copy(
                out_hbm.at[pl.ds(base, GRP)], grp_v.at[0], out_sem
            ).wait()

        def wave(p, lo):
            for lane in range(8):
                j = lo + lane

                @pl.when(mi[j] == 1)
                def _():
                    k128 = pl.multiple_of((vec[j] >> 7) * 128, 128)
                    pltpu.async_copy(
                        table_hbm.at[:, pl.ds(k128, LANES)],
                        blk_v.at[lane],
                        sem,
                    )

            for lane in range(8):
                j = lo + lane

                @pl.when(mi[j] == 1)
                def _():
                    pltpu.make_async_copy(
                        table_hbm.at[:, pl.ds(0, LANES)], blk_v.at[lane], sem
                    ).wait()

            for lane in range(8):
                j = lo + lane
                sv = jnp.full((16,), slot_vec[j], dtype=jnp.int32)
                q = jnp.full((16,), vec[j] & 127, dtype=jnp.int32)
                for t in range(EMBED_DIM // 16):
                    vals = plsc.load_gather(blk_v, [sv, row_ids[t], q])
                    grp_v.at[p].at[j].at[pl.ds(t * 16, 16)][...] = vals

        def group(p):
            wave(p, 0)
            wave(p, 8)
            for j in range(16):
                pltpu.async_copy(
                    grp_v.at[p].at[pl.ds(j, 1)],
                    out_hbm.at[pl.ds(ovec[j], 1)],
                    out_sem,
                )

        @pl.when(lax.rem(g, 2) == 0)
        def _():
            group(0)

        @pl.when(lax.rem(g, 2) == 1)
        def _():
            group(1)

        return carry

    lax.fori_loop(0, NGRP, body, None)
    for _ in range(2):
        pltpu.make_async_copy(
            out_hbm.at[pl.ds(base, GRP)], grp_v.at[0], out_sem
        ).wait()


def kernel(labels, embedding_table):
    iota = jnp.arange(BATCH, dtype=jnp.int32)
    slab, order = lax.sort((labels.astype(jnp.int32), iota), num_keys=1)
    return _gather_blocks(embedding_table.T, slab, order)


# speedup vs baseline: 3.4306x; 1.0296x over previous
import functools

import jax
import jax.numpy as jnp
from jax import lax
from jax.experimental import pallas as pl
from jax.experimental.pallas import tpu as pltpu
from jax.experimental.pallas import tpu_sc as plsc

NUM_CLASSES = 1000000
EMBED_DIM = 64
BATCH = 16384

NC = 2
NS = 16
NW = NC * NS

B_PER_W = BATCH // NW
GRP = 16
NGRP = B_PER_W // GRP
LANES = 128


@functools.partial(
    pl.kernel,
    mesh=plsc.VectorSubcoreMesh(core_axis_name="c", subcore_axis_name="s"),
    out_type=jax.ShapeDtypeStruct((BATCH, EMBED_DIM), jnp.float32),
    scratch_types=[
        pltpu.VMEM((B_PER_W,), jnp.int32),
        pltpu.VMEM((B_PER_W,), jnp.int32),
        pltpu.VMEM((8 * EMBED_DIM, LANES), jnp.float32),
        pltpu.VMEM((2, GRP, EMBED_DIM), jnp.float32),
        pltpu.SemaphoreType.DMA,
        pltpu.SemaphoreType.DMA,
    ],
    compiler_params=pltpu.CompilerParams(needs_layout_passes=False),
)
def _gather_blocks(table_hbm, idx_hbm, ord_hbm, out_hbm,
                   idx_v, ord_v, blk_v, grp_v, sem, out_sem):
    wid = lax.axis_index("s") * NC + lax.axis_index("c")
    base = wid * B_PER_W
    pltpu.sync_copy(idx_hbm.at[pl.ds(base, B_PER_W)], idx_v)
    pltpu.sync_copy(ord_hbm.at[pl.ds(base, B_PER_W)], ord_v)

    lane_ids = jnp.arange(16, dtype=jnp.int32)
    row_ids = [lane_ids + 16 * t for t in range(EMBED_DIM // 16)]

    def body(g, carry):
        vec = idx_v[pl.ds(g * GRP, GRP)]
        ovec = ord_v[pl.ds(g * GRP, GRP)]

        kvec = vec >> 7
        pidx = jnp.maximum(g * GRP + lane_ids - 1, 0)
        prev = plsc.load_gather(idx_v, [pidx]) >> 7
        m = (kvec != prev) | (lane_ids == 0)
        mi = m.astype(jnp.int32)
        slot_vec = (jnp.cumsum(mi) - 1) & 7

        @pl.when(g >= 2)
        def _():
            pltpu.make_async_copy(
                out_hbm.at[pl.ds(base, GRP)], grp_v.at[0], out_sem
            ).wait()

        def wave(p, lo):
            for lane in range(8):
                j = lo + lane

                @pl.when(mi[j] == 1)
                def _():
                    k128 = pl.multiple_of((vec[j] >> 7) * 128, 128)
                    pltpu.async_copy(
                        table_hbm.at[:, pl.ds(k128, LANES)],
                        blk_v.at[pl.ds(slot_vec[j] * EMBED_DIM, EMBED_DIM)],
                        sem,
                    )

            for lane in range(8):
                j = lo + lane

                @pl.when(mi[j] == 1)
                def _():
                    pltpu.make_async_copy(
                        table_hbm.at[:, pl.ds(0, LANES)],
                        blk_v.at[pl.ds(0, EMBED_DIM)],
                        sem,
                    ).wait()

            for lane in range(8):
                j = lo + lane
                sbase = jnp.full((16,), slot_vec[j] * EMBED_DIM, dtype=jnp.int32)
                q = jnp.full((16,), vec[j] & 127, dtype=jnp.int32)
                for t in range(EMBED_DIM // 16):
                    vals = plsc.load_gather(blk_v, [sbase + row_ids[t], q])
                    grp_v.at[p].at[j].at[pl.ds(t * 16, 16)][...] = vals

        def group(p):
            wave(p, 0)
            wave(p, 8)
            for j in range(16):
                pltpu.async_copy(
                    grp_v.at[p].at[pl.ds(j, 1)],
                    out_hbm.at[pl.ds(ovec[j], 1)],
                    out_sem,
                )

        @pl.when(lax.rem(g, 2) == 0)
        def _():
            group(0)

        @pl.when(lax.rem(g, 2) == 1)
        def _():
            group(1)

        return carry

    lax.fori_loop(0, NGRP, body, None)
    for _ in range(2):
        pltpu.make_async_copy(
            out_hbm.at[pl.ds(base, GRP)], grp_v.at[0], out_sem
        ).wait()


def kernel(labels, embedding_table):
    iota = jnp.arange(BATCH, dtype=jnp.int32)
    slab, order = lax.sort((labels.astype(jnp.int32), iota), num_keys=1)
    return _gather_blocks(embedding_table.T, slab, order)


# speedup vs baseline: 3.6590x; 1.0666x over previous
import functools

import jax
import jax.numpy as jnp
from jax import lax
from jax.experimental import pallas as pl
from jax.experimental.pallas import tpu as pltpu
from jax.experimental.pallas import tpu_sc as plsc

NUM_CLASSES = 1000000
EMBED_DIM = 64
BATCH = 16384

NC = 2
NS = 16
NW = NC * NS

B_PER_W = BATCH // NW
LANES = 128
CHUNK_K = 4
CW = CHUNK_K * LANES
NBLK = (NUM_CLASSES + LANES - 1) // LANES
MAXC = (NBLK + CHUNK_K - 1) // CHUNK_K + 1
MAX_K0 = NBLK - CHUNK_K


@functools.partial(
    pl.kernel,
    mesh=plsc.VectorSubcoreMesh(core_axis_name="c", subcore_axis_name="s"),
    out_type=jax.ShapeDtypeStruct((BATCH, EMBED_DIM), jnp.float32),
    scratch_types=[
        pltpu.VMEM((B_PER_W,), jnp.int32),
        pltpu.VMEM((B_PER_W,), jnp.int32),
        pltpu.VMEM((MAXC + 1,), jnp.int32),
        pltpu.VMEM((MAXC,), jnp.int32),
        pltpu.VMEM((16,), jnp.int32),
        pltpu.VMEM((2, EMBED_DIM, CW), jnp.float32),
        pltpu.VMEM((16, EMBED_DIM), jnp.float32),
        pltpu.SemaphoreType.DMA,
        pltpu.SemaphoreType.DMA,
        pltpu.SemaphoreType.DMA,
    ],
    compiler_params=pltpu.CompilerParams(needs_layout_passes=False),
)
def _gather_scan(table_hbm, idx_hbm, ord_hbm, s_hbm, ks_hbm, meta_hbm, out_hbm,
                 idx_v, ord_v, s_v, ks_v, meta_v, buf_v, ring_v,
                 sem_a, sem_b, out_sem):
    wid = lax.axis_index("s") * NC + lax.axis_index("c")
    base = wid * B_PER_W
    pltpu.sync_copy(idx_hbm.at[pl.ds(base, B_PER_W)], idx_v)
    pltpu.sync_copy(ord_hbm.at[pl.ds(base, B_PER_W)], ord_v)
    pltpu.sync_copy(s_hbm.at[wid], s_v)
    pltpu.sync_copy(ks_hbm.at[wid], ks_v)
    pltpu.sync_copy(meta_hbm.at[wid], meta_v)
    n_c = meta_v[pl.ds(0, 16)][0]

    lane_ids = jnp.arange(16, dtype=jnp.int32)
    row_ids = [lane_ids + 16 * t for t in range(EMBED_DIM // 16)]

    def kstart(c):
        v = plsc.load_gather(ks_v, [jnp.full((16,), c, jnp.int32)])
        return pl.multiple_of(v[0] * LANES, LANES)

    def svat(c):
        return plsc.load_gather(s_v, [jnp.full((16,), c, jnp.int32)])[0]

    def fire(c, p, sem):
        pltpu.async_copy(
            table_hbm.at[:, pl.ds(kstart(c), CW)], buf_v.at[p], sem
        )

    def drain(p, sem):
        pltpu.make_async_copy(
            table_hbm.at[:, pl.ds(0, CW)], buf_v.at[p], sem
        ).wait()

    def extract_window(c, p):
        ks0 = kstart(c)
        lo = svat(c)
        hi = svat(c + 1)

        def lab_body(i, carry):
            iv = jnp.full((16,), i, jnp.int32)
            lab = plsc.load_gather(idx_v, [iv])[0]
            bo = plsc.load_gather(ord_v, [iv])[0]
            col = jnp.full((16,), lab - ks0, jnp.int32)
            slot = lax.rem(i, 16)
            sl16 = jnp.full((16,), slot, jnp.int32)
            for t in range(EMBED_DIM // 16):
                vals = plsc.load_gather(buf_v.at[p], [row_ids[t], col])
                plsc.store_scatter(ring_v, [sl16, row_ids[t]], vals)

            @pl.when(i >= 16)
            def _():
                pltpu.make_async_copy(
                    out_hbm.at[pl.ds(base, 1)], ring_v.at[pl.ds(0, 1)], out_sem
                ).wait()

            pltpu.async_copy(
                ring_v.at[pl.ds(slot, 1)], out_hbm.at[pl.ds(bo, 1)], out_sem
            )
            return carry

        lax.fori_loop(lo, hi, lab_body, 0)

    fire(0, 0, sem_a)

    def body(c, carry):
        @pl.when(lax.rem(c, 2) == 0)
        def _():
            @pl.when(c + 1 < n_c)
            def _():
                fire(c + 1, 1, sem_b)

            drain(0, sem_a)
            extract_window(c, 0)

        @pl.when(lax.rem(c, 2) == 1)
        def _():
            @pl.when(c + 1 < n_c)
            def _():
                fire(c + 1, 0, sem_a)

            drain(1, sem_b)
            extract_window(c, 1)

        return carry

    lax.fori_loop(0, n_c, body, 0)

    for _ in range(16):
        pltpu.make_async_copy(
            out_hbm.at[pl.ds(base, 1)], ring_v.at[pl.ds(0, 1)], out_sem
        ).wait()


def kernel(labels, embedding_table):
    iota = jnp.arange(BATCH, dtype=jnp.int32)
    slab, order = lax.sort((labels.astype(jnp.int32), iota), num_keys=1)

    kcol = slab >> 7
    kt = kcol.reshape(NW, B_PER_W)
    kf = kt[:, 0]
    cl = (kt - kf[:, None]) >> 2
    n_c = cl[:, -1] + 1
    crange = jnp.arange(MAXC, dtype=jnp.int32)
    hist = jax.vmap(lambda c: jnp.bincount(c, length=MAXC))(cl)
    s = jnp.concatenate(
        [jnp.zeros((NW, 1), jnp.int32),
         jnp.cumsum(hist, axis=1, dtype=jnp.int32)], axis=1)
    ks = jnp.minimum(kf[:, None] + CHUNK_K * crange[None, :], MAX_K0)
    meta = jnp.broadcast_to(n_c[:, None], (NW, 16)).astype(jnp.int32)

    return _gather_scan(embedding_table.T, slab, order,
                        s.astype(jnp.int32), ks.astype(jnp.int32), meta)


# speedup vs baseline: 4.2501x; 1.1616x over previous
import functools

import jax
import jax.numpy as jnp
from jax import lax
from jax.experimental import pallas as pl
from jax.experimental.pallas import tpu as pltpu
from jax.experimental.pallas import tpu_sc as plsc

NUM_CLASSES = 1000000
EMBED_DIM = 64
BATCH = 16384

NC = 2
NS = 16
NW = NC * NS

B_PER_W = BATCH // NW
LANES = 128
CHUNK_K = 4
CW = CHUNK_K * LANES
NBLK = (NUM_CLASSES + LANES - 1) // LANES
MAX_K0 = NBLK - CHUNK_K


@functools.partial(
    pl.kernel,
    mesh=plsc.VectorSubcoreMesh(core_axis_name="c", subcore_axis_name="s"),
    out_type=jax.ShapeDtypeStruct((BATCH, EMBED_DIM), jnp.float32),
    scratch_types=[
        pltpu.VMEM((B_PER_W,), jnp.int32),
        pltpu.VMEM((B_PER_W,), jnp.int32),
        pltpu.VMEM((2, EMBED_DIM, CW), jnp.float32),
        pltpu.VMEM((16, EMBED_DIM), jnp.float32),
        pltpu.SemaphoreType.DMA,
        pltpu.SemaphoreType.DMA,
        pltpu.SemaphoreType.DMA,
    ],
    compiler_params=pltpu.CompilerParams(needs_layout_passes=False),
)
def _gather_scan(table_hbm, idx_hbm, ord_hbm, out_hbm,
                 idx_v, ord_v, buf_v, ring_v, sem_a, sem_b, out_sem):
    wid = lax.axis_index("s") * NC + lax.axis_index("c")
    base = wid * B_PER_W
    pltpu.sync_copy(idx_hbm.at[pl.ds(base, B_PER_W)], idx_v)
    pltpu.sync_copy(ord_hbm.at[pl.ds(base, B_PER_W)], ord_v)

    lane_ids = jnp.arange(16, dtype=jnp.int32)
    row_ids = [lane_ids + 16 * t for t in range(EMBED_DIM // 16)]
    zeros16 = jnp.zeros((16,), jnp.int32)

    def at(ref, i):
        return plsc.load_gather(ref, [jnp.full((16,), i, jnp.int32)])[0]

    kf = at(idx_v, 0) >> 7
    klast = at(idx_v, B_PER_W - 1) >> 7
    n_c = ((klast - kf) >> 2) + 1
    kf128 = kf * LANES

    def kstart(c):
        return pl.multiple_of(jnp.minimum(kf + CHUNK_K * c, MAX_K0) * LANES, LANES)

    def fire(c, p, sem):
        pltpu.async_copy(
            table_hbm.at[:, pl.ds(kstart(c), CW)], buf_v.at[p], sem
        )

    def drain(p, sem):
        pltpu.make_async_copy(
            table_hbm.at[:, pl.ds(0, CW)], buf_v.at[p], sem
        ).wait()

    def extract_window(c, p, ptr):
        ks0 = kstart(c)
        be = kf128 + (c + 1) * CW

        def cond(i):
            lab = at(idx_v, jnp.minimum(i, B_PER_W - 1))
            return (i < B_PER_W) & (lab < be)

        def lab_body(i):
            lab = at(idx_v, i)
            bo = at(ord_v, i)
            col = jnp.full((16,), lab - ks0, jnp.int32)
            slot = lax.rem(i, 16)
            sl16 = jnp.full((16,), slot, jnp.int32)
            for t in range(EMBED_DIM // 16):
                vals = plsc.load_gather(buf_v.at[p], [row_ids[t], col])
                plsc.store_scatter(ring_v, [sl16, row_ids[t]], vals)

            @pl.when(i >= 16)
            def _():
                pltpu.make_async_copy(
                    out_hbm.at[pl.ds(base, 1)], ring_v.at[pl.ds(0, 1)], out_sem
                ).wait()

            pltpu.async_copy(
                ring_v.at[pl.ds(slot, 1)], out_hbm.at[pl.ds(bo, 1)], out_sem
            )
            return i + 1

        return lax.while_loop(cond, lab_body, ptr)

    fire(0, 0, sem_a)

    def body(c, ptr):
        def even(ptr):
            @pl.when(c + 1 < n_c)
            def _():
                fire(c + 1, 1, sem_b)

            drain(0, sem_a)
            return extract_window(c, 0, ptr)

        def odd(ptr):
            @pl.when(c + 1 < n_c)
            def _():
                fire(c + 1, 0, sem_a)

            drain(1, sem_b)
            return extract_window(c, 1, ptr)

        return lax.cond(lax.rem(c, 2) == 0, even, odd, ptr)

    lax.fori_loop(0, n_c, body, 0)

    for _ in range(16):
        pltpu.make_async_copy(
            out_hbm.at[pl.ds(base, 1)], ring_v.at[pl.ds(0, 1)], out_sem
        ).wait()


def kernel(labels, embedding_table):
    iota = jnp.arange(BATCH, dtype=jnp.int32)
    slab, order = lax.sort((labels.astype(jnp.int32), iota), num_keys=1)
    return _gather_scan(embedding_table.T, slab, order)
